# initial kernel scaffold (unmeasured)
import jax
import jax.numpy as jnp
from jax import lax
from jax.experimental import pallas as pl
from jax.experimental.pallas import tpu as pltpu


def kernel(
    x,
):
    def body(*refs):
        pass

    out_shape = jax.ShapeDtypeStruct(..., jnp.float32)
    return pl.pallas_call(body, out_shape=out_shape)(...)


# baseline (device time: 84487 ns/iter reference)
import jax
import jax.numpy as jnp
from jax import lax
from jax.experimental import pallas as pl
from jax.experimental.pallas import tpu as pltpu

N_DEV = 4
M = 2048
N_OUT = 512


def kernel(x):
    xb = x[0].astype(jnp.bfloat16)

    def body(x_ref, out_ref, send_buf, recv_buf, send_sems, recv_sems):
        my_x = lax.axis_index("x")
        my_y = lax.axis_index("y")
        my_z = lax.axis_index("z")
        left = (my_z + N_DEV - 1) % N_DEV
        right = (my_z + 1) % N_DEV

        barrier_sem = pltpu.get_barrier_semaphore()
        for nbr in (left, right):
            pl.semaphore_signal(
                barrier_sem, inc=1,
                device_id=(my_x, my_y, nbr),
                device_id_type=pl.DeviceIdType.MESH,
            )
        pl.semaphore_wait(barrier_sem, 2)

        def chunk(c):
            return x_ref[:, pl.ds(c * N_OUT, N_OUT)]

        send_buf[0] = chunk((my_z + N_DEV - 1) % N_DEV)
        for s in range(N_DEV - 1):
            rdma = pltpu.make_async_remote_copy(
                src_ref=send_buf.at[s],
                dst_ref=recv_buf.at[s],
                send_sem=send_sems.at[s],
                recv_sem=recv_sems.at[s],
                device_id=(my_x, my_y, right),
                device_id_type=pl.DeviceIdType.MESH,
            )
            rdma.start()
            rdma.wait()
            c = (my_z + 2 * N_DEV - 2 - s) % N_DEV
            if s < N_DEV - 2:
                send_buf[s + 1] = recv_buf[s] + chunk(c)
            else:
                out_ref[:, :] = (
                    recv_buf[s].astype(jnp.float32)
                    + chunk(c).astype(jnp.float32)
                )

    return pl.pallas_call(
        body,
        out_shape=jax.ShapeDtypeStruct((M, N_OUT), jnp.float32),
        in_specs=[pl.BlockSpec(memory_space=pltpu.VMEM)],
        out_specs=pl.BlockSpec(memory_space=pltpu.VMEM),
        scratch_shapes=[
            pltpu.VMEM((N_DEV - 1, M, N_OUT), jnp.bfloat16),
            pltpu.VMEM((N_DEV - 1, M, N_OUT), jnp.bfloat16),
            pltpu.SemaphoreType.DMA((N_DEV - 1,)),
            pltpu.SemaphoreType.DMA((N_DEV - 1,)),
        ],
        compiler_params=pltpu.CompilerParams(collective_id=0),
    )(xb)


# device time: 84143 ns/iter; 1.0041x vs baseline; 1.0041x over previous
import jax
import jax.numpy as jnp
from jax import lax
from jax.experimental import pallas as pl
from jax.experimental.pallas import tpu as pltpu

N_DEV = 4
M = 2048
MH = M // 2
N_OUT = 512


def kernel(x):
    xb = x[0].astype(jnp.bfloat16)

    def body(x_ref, out_ref,
             send_cw, recv_cw, send_ccw, recv_ccw,
             send_sems_cw, recv_sems_cw, send_sems_ccw, recv_sems_ccw):
        my_x = lax.axis_index("x")
        my_y = lax.axis_index("y")
        my_z = lax.axis_index("z")
        left = (my_z + N_DEV - 1) % N_DEV
        right = (my_z + 1) % N_DEV

        barrier_sem = pltpu.get_barrier_semaphore()
        for nbr in (left, right):
            pl.semaphore_signal(
                barrier_sem, inc=1,
                device_id=(my_x, my_y, nbr),
                device_id_type=pl.DeviceIdType.MESH,
            )
        pl.semaphore_wait(barrier_sem, 2)

        def chunk_hi(c):
            return x_ref[pl.ds(0, MH), pl.ds(c * N_OUT, N_OUT)]

        def chunk_lo(c):
            return x_ref[pl.ds(MH, MH), pl.ds(c * N_OUT, N_OUT)]

        send_cw[0] = chunk_hi((my_z + N_DEV - 1) % N_DEV)
        send_ccw[0] = chunk_lo((my_z + 1) % N_DEV)

        for s in range(N_DEV - 1):
            rdma_cw = pltpu.make_async_remote_copy(
                src_ref=send_cw.at[s],
                dst_ref=recv_cw.at[s],
                send_sem=send_sems_cw.at[s],
                recv_sem=recv_sems_cw.at[s],
                device_id=(my_x, my_y, right),
                device_id_type=pl.DeviceIdType.MESH,
            )
            rdma_ccw = pltpu.make_async_remote_copy(
                src_ref=send_ccw.at[s],
                dst_ref=recv_ccw.at[s],
                send_sem=send_sems_ccw.at[s],
                recv_sem=recv_sems_ccw.at[s],
                device_id=(my_x, my_y, left),
                device_id_type=pl.DeviceIdType.MESH,
            )
            rdma_cw.start()
            rdma_ccw.start()

            c_cw = (my_z + 2 * N_DEV - 2 - s) % N_DEV
            c_ccw = (my_z + 2 + s) % N_DEV

            rdma_cw.wait_recv()
            if s < N_DEV - 2:
                send_cw[s + 1] = recv_cw[s] + chunk_hi(c_cw)
            else:
                out_ref[pl.ds(0, MH), :] = (
                    recv_cw[s].astype(jnp.float32)
                    + chunk_hi(c_cw).astype(jnp.float32)
                )

            rdma_ccw.wait_recv()
            if s < N_DEV - 2:
                send_ccw[s + 1] = recv_ccw[s] + chunk_lo(c_ccw)
            else:
                out_ref[pl.ds(MH, MH), :] = (
                    recv_ccw[s].astype(jnp.float32)
                    + chunk_lo(c_ccw).astype(jnp.float32)
                )

            rdma_cw.wait_send()
            rdma_ccw.wait_send()

    return pl.pallas_call(
        body,
        out_shape=jax.ShapeDtypeStruct((M, N_OUT), jnp.float32),
        in_specs=[pl.BlockSpec(memory_space=pltpu.VMEM)],
        out_specs=pl.BlockSpec(memory_space=pltpu.VMEM),
        scratch_shapes=[
            pltpu.VMEM((N_DEV - 1, MH, N_OUT), jnp.bfloat16),
            pltpu.VMEM((N_DEV - 1, MH, N_OUT), jnp.bfloat16),
            pltpu.VMEM((N_DEV - 1, MH, N_OUT), jnp.bfloat16),
            pltpu.VMEM((N_DEV - 1, MH, N_OUT), jnp.bfloat16),
            pltpu.SemaphoreType.DMA((N_DEV - 1,)),
            pltpu.SemaphoreType.DMA((N_DEV - 1,)),
            pltpu.SemaphoreType.DMA((N_DEV - 1,)),
            pltpu.SemaphoreType.DMA((N_DEV - 1,)),
        ],
        compiler_params=pltpu.CompilerParams(collective_id=0),
    )(xb)
